# baseline (speedup 1.0000x reference)
import functools

import jax
import jax.numpy as jnp
from jax import lax
from jax.experimental import pallas as pl
from jax.experimental.pallas import tpu as pltpu
from jax.experimental.pallas import tpu_sc as plsc

N = 10000
E = 320000
D = 128
NC = 2
NS = 16
NPAD = 10240
TRASH = N
EPT = E // (NC * NS)
K = 80
CHUNKS = EPT // K
RPT = NPAD // NS

_mesh = plsc.VectorSubcoreMesh(
    core_axis_name="c", subcore_axis_name="s", num_cores=NC, num_subcores=NS
)


@functools.partial(
    pl.kernel,
    out_type=jax.ShapeDtypeStruct((NC, NPAD, D), jnp.float32),
    mesh=_mesh,
    scratch_types=[
        pltpu.VMEM((K,), jnp.int32),
        pltpu.VMEM((K,), jnp.int32),
        pltpu.VMEM((K, D), jnp.float32),
        pltpu.VMEM_SHARED((NPAD, D), jnp.float32),
        pltpu.SemaphoreType.DMA,
    ],
)
def _sc_agg(zp, row, colp, zeros, out, rowbuf, colbuf, msg, acc, sem):
    c = lax.axis_index("c")
    s = lax.axis_index("s")
    base = s * RPT
    pltpu.sync_copy(zeros.at[pl.ds(base, RPT)], acc.at[pl.ds(base, RPT)])
    plsc.subcore_barrier()

    ebase = (c * NS + s) * EPT

    def _body(g, carry):
        off = ebase + g * K
        pltpu.sync_copy(row.at[pl.ds(off, K)], rowbuf)
        pltpu.sync_copy(colp.at[pl.ds(off, K)], colbuf)
        pltpu.async_copy(zp.at[rowbuf], msg, sem).wait()
        pltpu.sync_copy(msg, acc.at[colbuf], add=True)
        return carry

    lax.fori_loop(0, CHUNKS, _body, 0)
    plsc.subcore_barrier()
    pltpu.sync_copy(acc.at[pl.ds(base, RPT)], out.at[c, pl.ds(base, RPT)])


@functools.partial(
    pl.kernel,
    out_type=jax.ShapeDtypeStruct((NC, NPAD, D), jnp.float32),
    mesh=_mesh,
    scratch_types=[
        pltpu.VMEM((K,), jnp.int32),
        pltpu.VMEM((K, D), jnp.float32),
        pltpu.VMEM_SHARED((NPAD, D), jnp.float32),
    ],
)
def _sc_deg(colp, zeros, ones, out, colbuf, buf, acc):
    c = lax.axis_index("c")
    s = lax.axis_index("s")
    base = s * RPT
    pltpu.sync_copy(zeros.at[pl.ds(base, RPT)], acc.at[pl.ds(base, RPT)])
    pltpu.sync_copy(ones, buf)
    plsc.subcore_barrier()

    ebase = (c * NS + s) * EPT

    def _body(g, carry):
        pltpu.sync_copy(colp.at[pl.ds(ebase + g * K, K)], colbuf)
        pltpu.sync_copy(buf, acc.at[colbuf], add=True)
        return carry

    lax.fori_loop(0, CHUNKS, _body, 0)
    plsc.subcore_barrier()
    pltpu.sync_copy(acc.at[pl.ds(base, RPT)], out.at[c, pl.ds(base, RPT)])


_RB = 400
_GRID = N // _RB
_BE = 6400


def _tc_prep_body(e_ref, colp_ref):
    r = e_ref[0:1, :]
    c = e_ref[1:2, :]
    colp_ref[...] = jnp.where(r == c, TRASH, c)


_tc_prep = pl.pallas_call(
    _tc_prep_body,
    grid=(E // _BE,),
    in_specs=[pl.BlockSpec((2, _BE), lambda i: (0, i))],
    out_specs=pl.BlockSpec((1, _BE), lambda i: (0, i)),
    out_shape=jax.ShapeDtypeStruct((1, E), jnp.int32),
)


def _row_spec(width):
    return pl.BlockSpec((_RB, width), lambda i: (i, 0))


def _tc_first_body(x_ref, w_ref, d0_ref, d1_ref, zp_ref, dis_ref):
    deg = d0_ref[:, :1] + d1_ref[:, :1] + 1.0
    dis = lax.rsqrt(deg)
    z = jnp.dot(x_ref[...], w_ref[...], preferred_element_type=jnp.float32)
    zp_ref[...] = dis * z
    dis_ref[...] = jnp.broadcast_to(dis, (_RB, 16))


_tc_first = pl.pallas_call(
    _tc_first_body,
    grid=(_GRID,),
    in_specs=[
        _row_spec(D),
        pl.BlockSpec((D, D), lambda i: (0, 0)),
        _row_spec(D),
        _row_spec(D),
    ],
    out_specs=[_row_spec(D), _row_spec(16)],
    out_shape=[
        jax.ShapeDtypeStruct((N, D), jnp.float32),
        jax.ShapeDtypeStruct((N, 16), jnp.float32),
    ],
)


def _tc_mid_body(a0_ref, a1_ref, zp_ref, dis_ref, b_ref, w_ref, out_ref):
    dis = dis_ref[:, :1]
    h = dis * (a0_ref[...] + a1_ref[...] + zp_ref[...]) + b_ref[...]
    h = jnp.maximum(h, 0.0)
    out_ref[...] = dis * jnp.dot(
        h, w_ref[...], preferred_element_type=jnp.float32
    )


_tc_mid = pl.pallas_call(
    _tc_mid_body,
    grid=(_GRID,),
    in_specs=[
        _row_spec(D),
        _row_spec(D),
        _row_spec(D),
        _row_spec(16),
        pl.BlockSpec((1, D), lambda i: (0, 0)),
        pl.BlockSpec((D, D), lambda i: (0, 0)),
    ],
    out_specs=_row_spec(D),
    out_shape=jax.ShapeDtypeStruct((N, D), jnp.float32),
)


def _tc_last_body(a0_ref, a1_ref, zp_ref, dis_ref, b_ref, out_ref):
    dis = dis_ref[:, :1]
    out_ref[...] = dis * (a0_ref[...] + a1_ref[...] + zp_ref[...]) + b_ref[...]


_tc_last = pl.pallas_call(
    _tc_last_body,
    grid=(_GRID,),
    in_specs=[
        _row_spec(D),
        _row_spec(D),
        _row_spec(D),
        _row_spec(16),
        pl.BlockSpec((1, D), lambda i: (0, 0)),
    ],
    out_specs=_row_spec(D),
    out_shape=jax.ShapeDtypeStruct((N, D), jnp.float32),
)


def kernel(x, edge_index, W1, b1, W2, b2, W3, b3):
    row = edge_index[0]
    colp = _tc_prep(edge_index)[0]
    zeros = jnp.zeros((NPAD, D), jnp.float32)
    ones = jnp.ones((K, D), jnp.float32)

    degp = _sc_deg(colp, zeros, ones)
    z1p, dis16 = _tc_first(x, W1, degp[0, :N], degp[1, :N])

    a1 = _sc_agg(z1p, row, colp, zeros)
    z2p = _tc_mid(a1[0, :N], a1[1, :N], z1p, dis16, b1.reshape(1, D), W2)

    a2 = _sc_agg(z2p, row, colp, zeros)
    z3p = _tc_mid(a2[0, :N], a2[1, :N], z2p, dis16, b2.reshape(1, D), W3)

    a3 = _sc_agg(z3p, row, colp, zeros)
    return _tc_last(a3[0, :N], a3[1, :N], z3p, dis16, b3.reshape(1, D))


# speedup vs baseline: 1.3946x; 1.3946x over previous
import functools

import jax
import jax.numpy as jnp
from jax import lax
from jax.experimental import pallas as pl
from jax.experimental.pallas import tpu as pltpu
from jax.experimental.pallas import tpu_sc as plsc

N = 10000
E = 320000
D = 128
NC = 2
NS = 16
NPAD = 10240
TRASH = N
EPT = E // (NC * NS)
K = 80
CHUNKS = EPT // K
RPT = NPAD // NS

_mesh = plsc.VectorSubcoreMesh(
    core_axis_name="c", subcore_axis_name="s", num_cores=NC, num_subcores=NS
)


def _copy_chunk(src_all, dst, off):
    for j in range(K // 16):
        dst[pl.ds(j * 16, 16)] = src_all[pl.ds(off + j * 16, 16)]


@functools.partial(
    pl.kernel,
    out_type=jax.ShapeDtypeStruct((NC, NPAD, D), jnp.float32),
    mesh=_mesh,
    scratch_types=[
        pltpu.VMEM((K,), jnp.int32),
        pltpu.VMEM((K,), jnp.int32),
        pltpu.VMEM((EPT,), jnp.int32),
        pltpu.VMEM((EPT,), jnp.int32),
        pltpu.VMEM((K, D), jnp.float32),
        pltpu.VMEM_SHARED((NPAD, D), jnp.float32),
        pltpu.SemaphoreType.DMA,
    ],
)
def _sc_agg(zp, row, colp, zeros, out, rowbuf, colbuf, row_all, colp_all,
            msg, acc, sem):
    c = lax.axis_index("c")
    s = lax.axis_index("s")
    base = s * RPT
    pltpu.sync_copy(zeros.at[pl.ds(base, RPT)], acc.at[pl.ds(base, RPT)])
    ebase = (c * NS + s) * EPT
    pltpu.sync_copy(row.at[pl.ds(ebase, EPT)], row_all)
    pltpu.sync_copy(colp.at[pl.ds(ebase, EPT)], colp_all)
    plsc.subcore_barrier()

    def _body(g, carry):
        off = g * K
        _copy_chunk(row_all, rowbuf, off)
        _copy_chunk(colp_all, colbuf, off)
        pltpu.async_copy(zp.at[rowbuf], msg, sem).wait()
        pltpu.sync_copy(msg, acc.at[colbuf], add=True)
        return carry

    lax.fori_loop(0, CHUNKS, _body, 0)
    plsc.subcore_barrier()
    pltpu.sync_copy(acc.at[pl.ds(base, RPT)], out.at[c, pl.ds(base, RPT)])


@functools.partial(
    pl.kernel,
    out_type=jax.ShapeDtypeStruct((NC, NPAD, D), jnp.float32),
    mesh=_mesh,
    scratch_types=[
        pltpu.VMEM((K,), jnp.int32),
        pltpu.VMEM((EPT,), jnp.int32),
        pltpu.VMEM((K, D), jnp.float32),
        pltpu.VMEM_SHARED((NPAD, D), jnp.float32),
    ],
)
def _sc_deg(colp, zeros, ones, out, colbuf, colp_all, buf, acc):
    c = lax.axis_index("c")
    s = lax.axis_index("s")
    base = s * RPT
    pltpu.sync_copy(zeros.at[pl.ds(base, RPT)], acc.at[pl.ds(base, RPT)])
    pltpu.sync_copy(ones, buf)
    ebase = (c * NS + s) * EPT
    pltpu.sync_copy(colp.at[pl.ds(ebase, EPT)], colp_all)
    plsc.subcore_barrier()

    def _body(g, carry):
        _copy_chunk(colp_all, colbuf, g * K)
        pltpu.sync_copy(buf, acc.at[colbuf], add=True)
        return carry

    lax.fori_loop(0, CHUNKS, _body, 0)
    plsc.subcore_barrier()
    pltpu.sync_copy(acc.at[pl.ds(base, RPT)], out.at[c, pl.ds(base, RPT)])


_RB = 400
_GRID = N // _RB
_BE = 6400


def _tc_prep_body(e_ref, colp_ref):
    r = e_ref[0:1, :]
    c = e_ref[1:2, :]
    colp_ref[...] = jnp.where(r == c, TRASH, c)


_tc_prep = pl.pallas_call(
    _tc_prep_body,
    grid=(E // _BE,),
    in_specs=[pl.BlockSpec((2, _BE), lambda i: (0, i))],
    out_specs=pl.BlockSpec((1, _BE), lambda i: (0, i)),
    out_shape=jax.ShapeDtypeStruct((1, E), jnp.int32),
)


def _row_spec(width):
    return pl.BlockSpec((_RB, width), lambda i: (i, 0))


def _tc_first_body(x_ref, w_ref, d0_ref, d1_ref, zp_ref, dis_ref):
    deg = d0_ref[:, :1] + d1_ref[:, :1] + 1.0
    dis = lax.rsqrt(deg)
    z = jnp.dot(x_ref[...], w_ref[...], preferred_element_type=jnp.float32)
    zp_ref[...] = dis * z
    dis_ref[...] = jnp.broadcast_to(dis, (_RB, 16))


_tc_first = pl.pallas_call(
    _tc_first_body,
    grid=(_GRID,),
    in_specs=[
        _row_spec(D),
        pl.BlockSpec((D, D), lambda i: (0, 0)),
        _row_spec(D),
        _row_spec(D),
    ],
    out_specs=[_row_spec(D), _row_spec(16)],
    out_shape=[
        jax.ShapeDtypeStruct((N, D), jnp.float32),
        jax.ShapeDtypeStruct((N, 16), jnp.float32),
    ],
)


def _tc_mid_body(a0_ref, a1_ref, zp_ref, dis_ref, b_ref, w_ref, out_ref):
    dis = dis_ref[:, :1]
    h = dis * (a0_ref[...] + a1_ref[...] + zp_ref[...]) + b_ref[...]
    h = jnp.maximum(h, 0.0)
    out_ref[...] = dis * jnp.dot(
        h, w_ref[...], preferred_element_type=jnp.float32
    )


_tc_mid = pl.pallas_call(
    _tc_mid_body,
    grid=(_GRID,),
    in_specs=[
        _row_spec(D),
        _row_spec(D),
        _row_spec(D),
        _row_spec(16),
        pl.BlockSpec((1, D), lambda i: (0, 0)),
        pl.BlockSpec((D, D), lambda i: (0, 0)),
    ],
    out_specs=_row_spec(D),
    out_shape=jax.ShapeDtypeStruct((N, D), jnp.float32),
)


def _tc_last_body(a0_ref, a1_ref, zp_ref, dis_ref, b_ref, out_ref):
    dis = dis_ref[:, :1]
    out_ref[...] = dis * (a0_ref[...] + a1_ref[...] + zp_ref[...]) + b_ref[...]


_tc_last = pl.pallas_call(
    _tc_last_body,
    grid=(_GRID,),
    in_specs=[
        _row_spec(D),
        _row_spec(D),
        _row_spec(D),
        _row_spec(16),
        pl.BlockSpec((1, D), lambda i: (0, 0)),
    ],
    out_specs=_row_spec(D),
    out_shape=jax.ShapeDtypeStruct((N, D), jnp.float32),
)


def kernel(x, edge_index, W1, b1, W2, b2, W3, b3):
    row = edge_index[0]
    colp = _tc_prep(edge_index)[0]
    zeros = jnp.zeros((NPAD, D), jnp.float32)
    ones = jnp.ones((K, D), jnp.float32)

    degp = _sc_deg(colp, zeros, ones)
    z1p, dis16 = _tc_first(x, W1, degp[0, :N], degp[1, :N])

    a1 = _sc_agg(z1p, row, colp, zeros)
    z2p = _tc_mid(a1[0, :N], a1[1, :N], z1p, dis16, b1.reshape(1, D), W2)

    a2 = _sc_agg(z2p, row, colp, zeros)
    z3p = _tc_mid(a2[0, :N], a2[1, :N], z2p, dis16, b2.reshape(1, D), W3)

    a3 = _sc_agg(z3p, row, colp, zeros)
    return _tc_last(a3[0, :N], a3[1, :N], z3p, dis16, b3.reshape(1, D))


# speedup vs baseline: 1.6896x; 1.2116x over previous
import functools

import jax
import jax.numpy as jnp
from jax import lax
from jax.experimental import pallas as pl
from jax.experimental.pallas import tpu as pltpu
from jax.experimental.pallas import tpu_sc as plsc

N = 10000
E = 320000
D = 128
NC = 2
NS = 16
NPAD = 10240
TRASH = N
EPT = E // (NC * NS)
K = 80
CHUNKS = EPT // K
RPT = NPAD // NS

_mesh = plsc.VectorSubcoreMesh(
    core_axis_name="c", subcore_axis_name="s", num_cores=NC, num_subcores=NS
)


def _copy_chunk(src_all, dst, off):
    for j in range(K // 16):
        dst[pl.ds(j * 16, 16)] = src_all[pl.ds(off + j * 16, 16)]


@functools.partial(
    pl.kernel,
    out_type=jax.ShapeDtypeStruct((NC, NPAD, D), jnp.float32),
    mesh=_mesh,
    scratch_types=[
        pltpu.VMEM((K,), jnp.int32),
        pltpu.VMEM((K,), jnp.int32),
        pltpu.VMEM((K,), jnp.int32),
        pltpu.VMEM((K,), jnp.int32),
        pltpu.VMEM((EPT,), jnp.int32),
        pltpu.VMEM((EPT,), jnp.int32),
        pltpu.VMEM((K, D), jnp.float32),
        pltpu.VMEM((K, D), jnp.float32),
        pltpu.VMEM_SHARED((NPAD, D), jnp.float32),
        pltpu.SemaphoreType.DMA,
        pltpu.SemaphoreType.DMA,
    ],
)
def _sc_agg(zp, row, colp, zeros, out, rowbufA, colbufA, rowbufB, colbufB,
            row_all, colp_all, msgA, msgB, acc, semA, semB):
    c = lax.axis_index("c")
    s = lax.axis_index("s")
    base = s * RPT
    pltpu.sync_copy(zeros.at[pl.ds(base, RPT)], acc.at[pl.ds(base, RPT)])
    ebase = (c * NS + s) * EPT
    pltpu.sync_copy(row.at[pl.ds(ebase, EPT)], row_all)
    pltpu.sync_copy(colp.at[pl.ds(ebase, EPT)], colp_all)
    plsc.subcore_barrier()

    def _stage(g, rb, cb):
        _copy_chunk(row_all, rb, g * K)
        _copy_chunk(colp_all, cb, g * K)

    def _scat(mb, cb):
        pltpu.sync_copy(mb, acc.at[cb], add=True)

    _stage(0, rowbufA, colbufA)
    pltpu.async_copy(zp.at[rowbufA], msgA, semA).wait()

    def _body(st, carry):
        g = 2 * st
        _stage(g + 1, rowbufB, colbufB)
        dB = pltpu.async_copy(zp.at[rowbufB], msgB, semB)
        _scat(msgA, colbufA)
        dB.wait()
        _stage(g + 2, rowbufA, colbufA)
        dA = pltpu.async_copy(zp.at[rowbufA], msgA, semA)
        _scat(msgB, colbufB)
        dA.wait()
        return carry

    lax.fori_loop(0, (CHUNKS - 1) // 2, _body, 0)
    _scat(msgA, colbufA)

    plsc.subcore_barrier()
    pltpu.sync_copy(acc.at[pl.ds(base, RPT)], out.at[c, pl.ds(base, RPT)])


@functools.partial(
    pl.kernel,
    out_type=jax.ShapeDtypeStruct((NC, NPAD, D), jnp.float32),
    mesh=_mesh,
    scratch_types=[
        pltpu.VMEM((K,), jnp.int32),
        pltpu.VMEM((EPT,), jnp.int32),
        pltpu.VMEM((K, D), jnp.float32),
        pltpu.VMEM_SHARED((NPAD, D), jnp.float32),
    ],
)
def _sc_deg(colp, zeros, ones, out, colbuf, colp_all, buf, acc):
    c = lax.axis_index("c")
    s = lax.axis_index("s")
    base = s * RPT
    pltpu.sync_copy(zeros.at[pl.ds(base, RPT)], acc.at[pl.ds(base, RPT)])
    pltpu.sync_copy(ones, buf)
    ebase = (c * NS + s) * EPT
    pltpu.sync_copy(colp.at[pl.ds(ebase, EPT)], colp_all)
    plsc.subcore_barrier()

    def _body(g, carry):
        _copy_chunk(colp_all, colbuf, g * K)
        pltpu.sync_copy(buf, acc.at[colbuf], add=True)
        return carry

    lax.fori_loop(0, CHUNKS, _body, 0)
    plsc.subcore_barrier()
    pltpu.sync_copy(acc.at[pl.ds(base, RPT)], out.at[c, pl.ds(base, RPT)])


_RB = 400
_GRID = N // _RB
_BE = 6400


def _tc_prep_body(e_ref, colp_ref):
    r = e_ref[0:1, :]
    c = e_ref[1:2, :]
    colp_ref[...] = jnp.where(r == c, TRASH, c)


_tc_prep = pl.pallas_call(
    _tc_prep_body,
    grid=(E // _BE,),
    in_specs=[pl.BlockSpec((2, _BE), lambda i: (0, i))],
    out_specs=pl.BlockSpec((1, _BE), lambda i: (0, i)),
    out_shape=jax.ShapeDtypeStruct((1, E), jnp.int32),
)


def _row_spec(width):
    return pl.BlockSpec((_RB, width), lambda i: (i, 0))


def _tc_first_body(x_ref, w_ref, d0_ref, d1_ref, zp_ref, dis_ref):
    deg = d0_ref[:, :1] + d1_ref[:, :1] + 1.0
    dis = lax.rsqrt(deg)
    z = jnp.dot(x_ref[...], w_ref[...], preferred_element_type=jnp.float32)
    zp_ref[...] = dis * z
    dis_ref[...] = jnp.broadcast_to(dis, (_RB, 16))


_tc_first = pl.pallas_call(
    _tc_first_body,
    grid=(_GRID,),
    in_specs=[
        _row_spec(D),
        pl.BlockSpec((D, D), lambda i: (0, 0)),
        _row_spec(D),
        _row_spec(D),
    ],
    out_specs=[_row_spec(D), _row_spec(16)],
    out_shape=[
        jax.ShapeDtypeStruct((N, D), jnp.float32),
        jax.ShapeDtypeStruct((N, 16), jnp.float32),
    ],
)


def _tc_mid_body(a0_ref, a1_ref, zp_ref, dis_ref, b_ref, w_ref, out_ref):
    dis = dis_ref[:, :1]
    h = dis * (a0_ref[...] + a1_ref[...] + zp_ref[...]) + b_ref[...]
    h = jnp.maximum(h, 0.0)
    out_ref[...] = dis * jnp.dot(
        h, w_ref[...], preferred_element_type=jnp.float32
    )


_tc_mid = pl.pallas_call(
    _tc_mid_body,
    grid=(_GRID,),
    in_specs=[
        _row_spec(D),
        _row_spec(D),
        _row_spec(D),
        _row_spec(16),
        pl.BlockSpec((1, D), lambda i: (0, 0)),
        pl.BlockSpec((D, D), lambda i: (0, 0)),
    ],
    out_specs=_row_spec(D),
    out_shape=jax.ShapeDtypeStruct((N, D), jnp.float32),
)


def _tc_last_body(a0_ref, a1_ref, zp_ref, dis_ref, b_ref, out_ref):
    dis = dis_ref[:, :1]
    out_ref[...] = dis * (a0_ref[...] + a1_ref[...] + zp_ref[...]) + b_ref[...]


_tc_last = pl.pallas_call(
    _tc_last_body,
    grid=(_GRID,),
    in_specs=[
        _row_spec(D),
        _row_spec(D),
        _row_spec(D),
        _row_spec(16),
        pl.BlockSpec((1, D), lambda i: (0, 0)),
    ],
    out_specs=_row_spec(D),
    out_shape=jax.ShapeDtypeStruct((N, D), jnp.float32),
)


def kernel(x, edge_index, W1, b1, W2, b2, W3, b3):
    row = edge_index[0]
    colp = _tc_prep(edge_index)[0]
    zeros = jnp.zeros((NPAD, D), jnp.float32)
    ones = jnp.ones((K, D), jnp.float32)

    degp = _sc_deg(colp, zeros, ones)
    z1p, dis16 = _tc_first(x, W1, degp[0, :N], degp[1, :N])

    a1 = _sc_agg(z1p, row, colp, zeros)
    z2p = _tc_mid(a1[0, :N], a1[1, :N], z1p, dis16, b1.reshape(1, D), W2)

    a2 = _sc_agg(z2p, row, colp, zeros)
    z3p = _tc_mid(a2[0, :N], a2[1, :N], z2p, dis16, b2.reshape(1, D), W3)

    a3 = _sc_agg(z3p, row, colp, zeros)
    return _tc_last(a3[0, :N], a3[1, :N], z3p, dis16, b3.reshape(1, D))


# speedup vs baseline: 1.6952x; 1.0033x over previous
import functools

import jax
import jax.numpy as jnp
from jax import lax
from jax.experimental import pallas as pl
from jax.experimental.pallas import tpu as pltpu
from jax.experimental.pallas import tpu_sc as plsc

N = 10000
E = 320000
D = 128
NC = 2
NS = 16
NPAD = 10240
TRASH = N
EPT = E // (NC * NS)
K = 80
CHUNKS = EPT // K
RPT = NPAD // NS

_mesh = plsc.VectorSubcoreMesh(
    core_axis_name="c", subcore_axis_name="s", num_cores=NC, num_subcores=NS
)


def _copy_chunk(src_all, dst, off):
    for j in range(K // 16):
        dst[pl.ds(j * 16, 16)] = src_all[pl.ds(off + j * 16, 16)]


@functools.partial(
    pl.kernel,
    out_type=jax.ShapeDtypeStruct((NC, NPAD, D), jnp.float32),
    mesh=_mesh,
    scratch_types=[
        pltpu.VMEM((K,), jnp.int32),
        pltpu.VMEM((K,), jnp.int32),
        pltpu.VMEM((K,), jnp.int32),
        pltpu.VMEM((K,), jnp.int32),
        pltpu.VMEM((EPT,), jnp.int32),
        pltpu.VMEM((EPT,), jnp.int32),
        pltpu.VMEM((K, D), jnp.float32),
        pltpu.VMEM((K, D), jnp.float32),
        pltpu.VMEM_SHARED((NPAD, D), jnp.float32),
        pltpu.SemaphoreType.DMA,
        pltpu.SemaphoreType.DMA,
    ],
)
def _sc_agg(zp, row, colp, zeros, out, rowbufA, colbufA, rowbufB, colbufB,
            row_all, colp_all, msgA, msgB, acc, semA, semB):
    c = lax.axis_index("c")
    s = lax.axis_index("s")
    base = s * RPT
    pltpu.sync_copy(zeros.at[pl.ds(base, RPT)], acc.at[pl.ds(base, RPT)])
    ebase = (c * NS + s) * EPT
    pltpu.sync_copy(row.at[pl.ds(ebase, EPT)], row_all)
    pltpu.sync_copy(colp.at[pl.ds(ebase, EPT)], colp_all)
    plsc.subcore_barrier()

    def _stage(g, rb, cb):
        _copy_chunk(row_all, rb, g * K)
        _copy_chunk(colp_all, cb, g * K)

    def _scat(mb, cb):
        pltpu.sync_copy(mb, acc.at[cb], add=True)

    _stage(0, rowbufA, colbufA)
    pltpu.async_copy(zp.at[rowbufA], msgA, semA).wait()

    def _body(st, carry):
        g = 2 * st
        _stage(g + 1, rowbufB, colbufB)
        dB = pltpu.async_copy(zp.at[rowbufB], msgB, semB)
        _scat(msgA, colbufA)
        dB.wait()
        _stage(g + 2, rowbufA, colbufA)
        dA = pltpu.async_copy(zp.at[rowbufA], msgA, semA)
        _scat(msgB, colbufB)
        dA.wait()
        return carry

    lax.fori_loop(0, (CHUNKS - 1) // 2, _body, 0)
    _scat(msgA, colbufA)

    plsc.subcore_barrier()
    pltpu.sync_copy(acc.at[pl.ds(base, RPT)], out.at[c, pl.ds(base, RPT)])


@functools.partial(
    pl.kernel,
    out_type=jax.ShapeDtypeStruct((NC, NPAD, D), jnp.float32),
    mesh=_mesh,
    scratch_types=[
        pltpu.VMEM((K,), jnp.int32),
        pltpu.VMEM((K,), jnp.int32),
        pltpu.VMEM((EPT,), jnp.int32),
        pltpu.VMEM((K, D), jnp.float32),
        pltpu.VMEM_SHARED((NPAD, D), jnp.float32),
        pltpu.SemaphoreType.DMA,
        pltpu.SemaphoreType.DMA,
    ],
)
def _sc_deg(colp, zeros, ones, out, colbufA, colbufB, colp_all, buf, acc,
            semA, semB):
    c = lax.axis_index("c")
    s = lax.axis_index("s")
    base = s * RPT
    pltpu.sync_copy(zeros.at[pl.ds(base, RPT)], acc.at[pl.ds(base, RPT)])
    pltpu.sync_copy(ones, buf)
    ebase = (c * NS + s) * EPT
    pltpu.sync_copy(colp.at[pl.ds(ebase, EPT)], colp_all)
    plsc.subcore_barrier()

    def _body(st, carry):
        g = 2 * st
        _copy_chunk(colp_all, colbufA, g * K)
        dA = pltpu.async_copy(buf, acc.at[colbufA], semA, add=True)
        _copy_chunk(colp_all, colbufB, (g + 1) * K)
        dB = pltpu.async_copy(buf, acc.at[colbufB], semB, add=True)
        dA.wait()
        dB.wait()
        return carry

    lax.fori_loop(0, CHUNKS // 2, _body, 0)
    _copy_chunk(colp_all, colbufA, (CHUNKS - 1) * K)
    pltpu.sync_copy(buf, acc.at[colbufA], add=True)
    plsc.subcore_barrier()
    pltpu.sync_copy(acc.at[pl.ds(base, RPT)], out.at[c, pl.ds(base, RPT)])


_RB = 400
_GRID = N // _RB
_BE = 6400


def _tc_prep_body(e_ref, colp_ref):
    r = e_ref[0:1, :]
    c = e_ref[1:2, :]
    colp_ref[...] = jnp.where(r == c, TRASH, c)


_tc_prep = pl.pallas_call(
    _tc_prep_body,
    grid=(E // _BE,),
    in_specs=[pl.BlockSpec((2, _BE), lambda i: (0, i))],
    out_specs=pl.BlockSpec((1, _BE), lambda i: (0, i)),
    out_shape=jax.ShapeDtypeStruct((1, E), jnp.int32),
)


def _row_spec(width):
    return pl.BlockSpec((_RB, width), lambda i: (i, 0))


def _tc_first_body(x_ref, w_ref, d0_ref, d1_ref, zp_ref, dis_ref):
    deg = d0_ref[:, :1] + d1_ref[:, :1] + 1.0
    dis = lax.rsqrt(deg)
    z = jnp.dot(x_ref[...], w_ref[...], preferred_element_type=jnp.float32)
    zp_ref[...] = dis * z
    dis_ref[...] = jnp.broadcast_to(dis, (_RB, 16))


_tc_first = pl.pallas_call(
    _tc_first_body,
    grid=(_GRID,),
    in_specs=[
        _row_spec(D),
        pl.BlockSpec((D, D), lambda i: (0, 0)),
        _row_spec(D),
        _row_spec(D),
    ],
    out_specs=[_row_spec(D), _row_spec(16)],
    out_shape=[
        jax.ShapeDtypeStruct((N, D), jnp.float32),
        jax.ShapeDtypeStruct((N, 16), jnp.float32),
    ],
)


def _tc_mid_body(a0_ref, a1_ref, zp_ref, dis_ref, b_ref, w_ref, out_ref):
    dis = dis_ref[:, :1]
    h = dis * (a0_ref[...] + a1_ref[...] + zp_ref[...]) + b_ref[...]
    h = jnp.maximum(h, 0.0)
    out_ref[...] = dis * jnp.dot(
        h, w_ref[...], preferred_element_type=jnp.float32
    )


_tc_mid = pl.pallas_call(
    _tc_mid_body,
    grid=(_GRID,),
    in_specs=[
        _row_spec(D),
        _row_spec(D),
        _row_spec(D),
        _row_spec(16),
        pl.BlockSpec((1, D), lambda i: (0, 0)),
        pl.BlockSpec((D, D), lambda i: (0, 0)),
    ],
    out_specs=_row_spec(D),
    out_shape=jax.ShapeDtypeStruct((N, D), jnp.float32),
)


def _tc_last_body(a0_ref, a1_ref, zp_ref, dis_ref, b_ref, out_ref):
    dis = dis_ref[:, :1]
    out_ref[...] = dis * (a0_ref[...] + a1_ref[...] + zp_ref[...]) + b_ref[...]


_tc_last = pl.pallas_call(
    _tc_last_body,
    grid=(_GRID,),
    in_specs=[
        _row_spec(D),
        _row_spec(D),
        _row_spec(D),
        _row_spec(16),
        pl.BlockSpec((1, D), lambda i: (0, 0)),
    ],
    out_specs=_row_spec(D),
    out_shape=jax.ShapeDtypeStruct((N, D), jnp.float32),
)


def kernel(x, edge_index, W1, b1, W2, b2, W3, b3):
    row = edge_index[0]
    colp = _tc_prep(edge_index)[0]
    zeros = jnp.zeros((NPAD, D), jnp.float32)
    ones = jnp.ones((K, D), jnp.float32)

    degp = _sc_deg(colp, zeros, ones)
    z1p, dis16 = _tc_first(x, W1, degp[0, :N], degp[1, :N])

    a1 = _sc_agg(z1p, row, colp, zeros)
    z2p = _tc_mid(a1[0, :N], a1[1, :N], z1p, dis16, b1.reshape(1, D), W2)

    a2 = _sc_agg(z2p, row, colp, zeros)
    z3p = _tc_mid(a2[0, :N], a2[1, :N], z2p, dis16, b2.reshape(1, D), W3)

    a3 = _sc_agg(z3p, row, colp, zeros)
    return _tc_last(a3[0, :N], a3[1, :N], z3p, dis16, b3.reshape(1, D))
